# Initial kernel scaffold; baseline (speedup 1.0000x reference)
import jax
import jax.numpy as jnp
from jax.experimental import pallas as pl


def kernel(weights, y, W_enc, codebooks):
    raise NotImplementedError("write your pallas kernel here")


# speedup vs baseline: 1.2201x; 1.2201x over previous
import functools

import jax
import jax.numpy as jnp
from jax import lax
from jax.experimental import pallas as pl
from jax.experimental.pallas import tpu as pltpu
from jax.experimental.pallas import tpu_sc as plsc

N_TOK = 2048
D_IN = 512
CODE_DIM = 256
K = 8192
NUM_Q = 4
KT = 1024
NSTEPS = K // KT


def _argmin_update(d, k, rn_ref, min_ref, arg_ref):
    m = jnp.min(d, axis=1, keepdims=True)
    iota = lax.broadcasted_iota(jnp.int32, d.shape, 1)
    cand = jnp.where(d == m, iota, jnp.int32(K))
    a = jnp.min(cand, axis=1, keepdims=True)
    better = m < min_ref[...]
    arg_ref[...] = jnp.where(better, a + k * KT, arg_ref[...])
    min_ref[...] = jnp.where(better, m, min_ref[...])


def _dist(r, cb, rn):
    s = lax.dot_general(r, cb, (((1,), (1,)), ((), ())),
                        preferred_element_type=jnp.float32)
    cbn = jnp.sum(cb * cb, axis=1)
    return rn - 2.0 * s + cbn[None, :]


def _first_body(w_ref, we_ref, cb_ref, idx_ref, r_ref, rn_ref, min_ref,
                arg_ref):
    k = pl.program_id(0)

    @pl.when(k == 0)
    def _():
        z = lax.dot_general(w_ref[...], we_ref[...],
                            (((1,), (0,)), ((), ())),
                            preferred_element_type=jnp.float32)
        r_ref[...] = z
        rn_ref[...] = jnp.sum(z * z, axis=1, keepdims=True)
        min_ref[...] = jnp.full((N_TOK, 1), jnp.inf, jnp.float32)
        arg_ref[...] = jnp.zeros((N_TOK, 1), jnp.int32)

    d = _dist(r_ref[...], cb_ref[...], rn_ref[...])
    _argmin_update(d, k, rn_ref, min_ref, arg_ref)

    @pl.when(k == NSTEPS - 1)
    def _():
        idx_ref[...] = arg_ref[...]


def _next_body(rp_ref, sel_ref, cb_ref, idx_ref, r_ref, rn_ref, min_ref,
               arg_ref):
    k = pl.program_id(0)

    @pl.when(k == 0)
    def _():
        r = rp_ref[...] - sel_ref[...]
        r_ref[...] = r
        rn_ref[...] = jnp.sum(r * r, axis=1, keepdims=True)
        min_ref[...] = jnp.full((N_TOK, 1), jnp.inf, jnp.float32)
        arg_ref[...] = jnp.zeros((N_TOK, 1), jnp.int32)

    d = _dist(r_ref[...], cb_ref[...], rn_ref[...])
    _argmin_update(d, k, rn_ref, min_ref, arg_ref)

    @pl.when(k == NSTEPS - 1)
    def _():
        idx_ref[...] = arg_ref[...]


_SCRATCH = [
    pltpu.VMEM((N_TOK, CODE_DIM), jnp.float32),
    pltpu.VMEM((N_TOK, 1), jnp.float32),
    pltpu.VMEM((N_TOK, 1), jnp.float32),
    pltpu.VMEM((N_TOK, 1), jnp.int32),
]

_CB_SPEC = pl.BlockSpec((KT, CODE_DIM), lambda k: (k, 0))


def _full_spec(shape):
    return pl.BlockSpec(shape, lambda k: tuple(0 for _ in shape))


def _vq_first(weights, W_enc, cb):
    idx, r = pl.pallas_call(
        _first_body,
        grid=(NSTEPS,),
        in_specs=[
            _full_spec((N_TOK, D_IN)),
            _full_spec((D_IN, CODE_DIM)),
            _CB_SPEC,
        ],
        out_specs=[
            _full_spec((N_TOK, 1)),
            _full_spec((N_TOK, CODE_DIM)),
        ],
        out_shape=[
            jax.ShapeDtypeStruct((N_TOK, 1), jnp.int32),
            jax.ShapeDtypeStruct((N_TOK, CODE_DIM), jnp.float32),
        ],
        scratch_shapes=_SCRATCH[1:],
        compiler_params=pltpu.CompilerParams(
            dimension_semantics=("arbitrary",)),
    )(weights, W_enc, cb)
    return idx, r


def _vq_next(r_prev, sel_prev, cb):
    idx, r = pl.pallas_call(
        _next_body,
        grid=(NSTEPS,),
        in_specs=[
            _full_spec((N_TOK, CODE_DIM)),
            _full_spec((N_TOK, CODE_DIM)),
            _CB_SPEC,
        ],
        out_specs=[
            _full_spec((N_TOK, 1)),
            _full_spec((N_TOK, CODE_DIM)),
        ],
        out_shape=[
            jax.ShapeDtypeStruct((N_TOK, 1), jnp.int32),
            jax.ShapeDtypeStruct((N_TOK, CODE_DIM), jnp.float32),
        ],
        scratch_shapes=_SCRATCH[1:],
        compiler_params=pltpu.CompilerParams(
            dimension_semantics=("arbitrary",)),
    )(r_prev, sel_prev, cb)
    return idx, r


def _sc_gather(table, idx):
    info = plsc.get_sparse_core_info()
    nw = info.num_cores * info.num_subcores
    b_per_w = N_TOK // nw
    mesh = plsc.VectorSubcoreMesh(core_axis_name="c", subcore_axis_name="s")

    @functools.partial(
        pl.kernel,
        mesh=mesh,
        out_type=jax.ShapeDtypeStruct((N_TOK, CODE_DIM), jnp.float32),
        scratch_types=[
            pltpu.VMEM((b_per_w,), jnp.int32),
            pltpu.VMEM((b_per_w, CODE_DIM), jnp.float32),
            pltpu.SemaphoreType.DMA,
        ],
    )
    def gather_kernel(table_hbm, idx_hbm, out_hbm, idx_v, rows_v, sem):
        wid = lax.axis_index("s") * info.num_cores + lax.axis_index("c")
        base = wid * b_per_w
        pltpu.sync_copy(idx_hbm.at[pl.ds(base, b_per_w)], idx_v)
        pltpu.async_copy(table_hbm.at[idx_v], rows_v, sem).wait()
        pltpu.sync_copy(rows_v, out_hbm.at[pl.ds(base, b_per_w)])

    return gather_kernel(table, idx)


def kernel(weights, y, W_enc, codebooks):
    cbs = [codebooks[q] for q in range(NUM_Q)]
    idx0, r = _vq_first(weights, W_enc, cbs[0])
    idx_cols = [idx0]
    for q in range(1, NUM_Q):
        sel = _sc_gather(cbs[q - 1], idx_cols[-1].reshape(N_TOK))
        idx_q, r = _vq_next(r, sel, cbs[q])
        idx_cols.append(idx_q)
    indices = jnp.concatenate(idx_cols, axis=1)
    bos = jnp.array([K], dtype=jnp.float32)
    eos = jnp.array([K + 1], dtype=jnp.float32)
    x = jnp.concatenate(
        [bos, indices.reshape(-1).astype(jnp.float32), eos])
    return (x, y)


# speedup vs baseline: 1.2637x; 1.0358x over previous
import functools

import jax
import jax.numpy as jnp
from jax import lax
from jax.experimental import pallas as pl
from jax.experimental.pallas import tpu as pltpu
from jax.experimental.pallas import tpu_sc as plsc

N_TOK = 2048
D_IN = 512
CODE_DIM = 256
K = 8192
NUM_Q = 4
NCORE = 2
TB = N_TOK // NCORE
KT = 1024
NSTEPS = K // KT


def _round_init(r, rm2_ref, rn_ref, min_ref, arg_ref):
    rm2_ref[...] = -2.0 * r
    rn_ref[...] = jnp.sum(r * r, axis=1, keepdims=True)
    min_ref[...] = jnp.full((TB, 1), jnp.inf, jnp.float32)
    arg_ref[...] = jnp.zeros((TB, 1), jnp.float32)


def _tile_update(k, cb_ref, rm2_ref, rn_ref, min_ref, arg_ref):
    cb = cb_ref[...]
    cbn = jnp.sum(cb * cb, axis=1)
    s2 = lax.dot_general(rm2_ref[...], cb, (((1,), (1,)), ((), ())),
                         preferred_element_type=jnp.float32)
    d = rn_ref[...] + s2 + cbn[None, :]
    m = jnp.min(d, axis=1, keepdims=True)
    iota = lax.broadcasted_iota(jnp.int32, d.shape, 1).astype(jnp.float32)
    cand = jnp.where(d == m, iota, jnp.float32(K))
    a = jnp.min(cand, axis=1, keepdims=True)
    better = m < min_ref[...]
    off = (k * KT).astype(jnp.float32)
    arg_ref[...] = jnp.where(better, a + off, arg_ref[...])
    min_ref[...] = jnp.where(better, m, min_ref[...])


def _emit_idx(k, idxf_ref, idxi_ref, arg_ref):
    @pl.when(k == NSTEPS - 1)
    def _():
        a = arg_ref[...]
        idxf_ref[...] = a
        if idxi_ref is not None:
            idxi_ref[...] = a.astype(jnp.int32)


def _first_body(w_ref, we_ref, cb_ref, idxf_ref, idxi_ref, r_ref,
                rm2_ref, rn_ref, min_ref, arg_ref):
    k = pl.program_id(1)

    @pl.when(k == 0)
    def _():
        z = lax.dot_general(w_ref[...], we_ref[...],
                            (((1,), (0,)), ((), ())),
                            preferred_element_type=jnp.float32)
        r_ref[...] = z
        _round_init(z, rm2_ref, rn_ref, min_ref, arg_ref)

    _tile_update(k, cb_ref, rm2_ref, rn_ref, min_ref, arg_ref)
    _emit_idx(k, idxf_ref, idxi_ref, arg_ref)


def _next_body(rp_ref, sel_ref, cb_ref, idxf_ref, idxi_ref, r_ref,
               rm2_ref, rn_ref, min_ref, arg_ref):
    k = pl.program_id(1)

    @pl.when(k == 0)
    def _():
        r = rp_ref[...] - sel_ref[...]
        r_ref[...] = r
        _round_init(r, rm2_ref, rn_ref, min_ref, arg_ref)

    _tile_update(k, cb_ref, rm2_ref, rn_ref, min_ref, arg_ref)
    _emit_idx(k, idxf_ref, idxi_ref, arg_ref)


def _last_body(rp_ref, sel_ref, cb_ref, idxf_ref,
               rm2_ref, rn_ref, min_ref, arg_ref):
    k = pl.program_id(1)

    @pl.when(k == 0)
    def _():
        r = rp_ref[...] - sel_ref[...]
        _round_init(r, rm2_ref, rn_ref, min_ref, arg_ref)

    _tile_update(k, cb_ref, rm2_ref, rn_ref, min_ref, arg_ref)
    _emit_idx(k, idxf_ref, None, arg_ref)


_SCRATCH = [
    pltpu.VMEM((TB, CODE_DIM), jnp.float32),
    pltpu.VMEM((TB, 1), jnp.float32),
    pltpu.VMEM((TB, 1), jnp.float32),
    pltpu.VMEM((TB, 1), jnp.float32),
]

_CB_SPEC = pl.BlockSpec((KT, CODE_DIM), lambda t, k: (k, 0))
_PARAMS = pltpu.CompilerParams(
    dimension_semantics=("parallel", "arbitrary"))


def _tok_spec(cols):
    return pl.BlockSpec((TB, cols), lambda t, k: (t, 0))


def _rep_spec(shape):
    return pl.BlockSpec(shape, lambda t, k: tuple(0 for _ in shape))


_IDX_OUT = [
    jax.ShapeDtypeStruct((N_TOK, 1), jnp.float32),
    jax.ShapeDtypeStruct((N_TOK, 1), jnp.int32),
]


def _vq_first(weights, W_enc, cb):
    return pl.pallas_call(
        _first_body,
        grid=(NCORE, NSTEPS),
        in_specs=[_tok_spec(D_IN), _rep_spec((D_IN, CODE_DIM)), _CB_SPEC],
        out_specs=[_tok_spec(1), _tok_spec(1), _tok_spec(CODE_DIM)],
        out_shape=_IDX_OUT + [
            jax.ShapeDtypeStruct((N_TOK, CODE_DIM), jnp.float32)],
        scratch_shapes=_SCRATCH,
        compiler_params=_PARAMS,
    )(weights, W_enc, cb)


def _vq_next(r_prev, sel_prev, cb):
    return pl.pallas_call(
        _next_body,
        grid=(NCORE, NSTEPS),
        in_specs=[_tok_spec(CODE_DIM), _tok_spec(CODE_DIM), _CB_SPEC],
        out_specs=[_tok_spec(1), _tok_spec(1), _tok_spec(CODE_DIM)],
        out_shape=_IDX_OUT + [
            jax.ShapeDtypeStruct((N_TOK, CODE_DIM), jnp.float32)],
        scratch_shapes=_SCRATCH,
        compiler_params=_PARAMS,
    )(r_prev, sel_prev, cb)


def _vq_last(r_prev, sel_prev, cb):
    return pl.pallas_call(
        _last_body,
        grid=(NCORE, NSTEPS),
        in_specs=[_tok_spec(CODE_DIM), _tok_spec(CODE_DIM), _CB_SPEC],
        out_specs=[_tok_spec(1)],
        out_shape=[jax.ShapeDtypeStruct((N_TOK, 1), jnp.float32)],
        scratch_shapes=_SCRATCH,
        compiler_params=_PARAMS,
    )(r_prev, sel_prev, cb)


def _sc_gather(table, idx):
    info = plsc.get_sparse_core_info()
    nw = info.num_cores * info.num_subcores
    b_per_w = N_TOK // nw
    mesh = plsc.VectorSubcoreMesh(core_axis_name="c", subcore_axis_name="s")

    @functools.partial(
        pl.kernel,
        mesh=mesh,
        out_type=jax.ShapeDtypeStruct((N_TOK, CODE_DIM), jnp.float32),
        scratch_types=[
            pltpu.VMEM((b_per_w,), jnp.int32),
            pltpu.VMEM((b_per_w, CODE_DIM), jnp.float32),
            pltpu.SemaphoreType.DMA,
        ],
    )
    def gather_kernel(table_hbm, idx_hbm, out_hbm, idx_v, rows_v, sem):
        wid = lax.axis_index("s") * info.num_cores + lax.axis_index("c")
        base = wid * b_per_w
        pltpu.sync_copy(idx_hbm.at[pl.ds(base, b_per_w)], idx_v)
        pltpu.async_copy(table_hbm.at[idx_v], rows_v, sem).wait()
        pltpu.sync_copy(rows_v, out_hbm.at[pl.ds(base, b_per_w)])

    return gather_kernel(table, idx)


def kernel(weights, y, W_enc, codebooks):
    cbs = [codebooks[q] for q in range(NUM_Q)]
    idxf0, idxi, r = _vq_first(weights, W_enc, cbs[0])
    idx_cols = [idxf0]
    for q in range(1, NUM_Q):
        sel = _sc_gather(cbs[q - 1], idxi.reshape(N_TOK))
        if q < NUM_Q - 1:
            idxf, idxi, r = _vq_next(r, sel, cbs[q])
        else:
            (idxf,) = _vq_last(r, sel, cbs[q])
        idx_cols.append(idxf)
    indices = jnp.concatenate(idx_cols, axis=1)
    bos = jnp.array([K], dtype=jnp.float32)
    eos = jnp.array([K + 1], dtype=jnp.float32)
    x = jnp.concatenate([bos, indices.reshape(-1), eos])
    return (x, y)


# speedup vs baseline: 1.4285x; 1.1304x over previous
import functools

import jax
import jax.numpy as jnp
from jax import lax
from jax.experimental import pallas as pl
from jax.experimental.pallas import tpu as pltpu
from jax.experimental.pallas import tpu_sc as plsc

N_TOK = 2048
D_IN = 512
CODE_DIM = 256
K = 8192
NUM_Q = 4
NBLK = 2
TB = N_TOK // NBLK
KT = 1024
NSTEPS = K // KT


def _round_init(r, rm2_ref, rn_ref, min_ref, arg_ref):
    rm2_ref[...] = -2.0 * r
    rn_ref[...] = jnp.sum(r * r, axis=1, keepdims=True)
    min_ref[...] = jnp.full((TB, 1), jnp.inf, jnp.float32)
    arg_ref[...] = jnp.zeros((TB, 1), jnp.float32)


def _tile_update(k, cb, rm2_ref, rn_ref, min_ref, arg_ref):
    cbn = jnp.sum(cb * cb, axis=1)
    s2 = lax.dot_general(rm2_ref[...], cb, (((1,), (1,)), ((), ())),
                         preferred_element_type=jnp.float32)
    d = rn_ref[...] + s2 + cbn[None, :]
    m = jnp.min(d, axis=1, keepdims=True)
    iota = lax.broadcasted_iota(jnp.int32, d.shape, 1).astype(jnp.float32)
    cand = jnp.where(d == m, iota, jnp.float32(K))
    a = jnp.min(cand, axis=1, keepdims=True)
    better = m < min_ref[...]
    off = (k * KT).astype(jnp.float32)
    arg_ref[...] = jnp.where(better, a + off, arg_ref[...])
    min_ref[...] = jnp.where(better, m, min_ref[...])


def _emit_idx(k, q, idxf_ref, idxi_ref, arg_ref):
    @pl.when(k == NSTEPS - 1)
    def _():
        a = arg_ref[...]
        idxf_ref[...] = a
        if idxi_ref is not None:
            idxi_ref[...] = a.astype(jnp.int32) + jnp.int32(q * K)


def _first_body(q, w_ref, we_ref, cb_ref, idxf_ref, idxi_ref, r_ref,
                rm2_ref, rn_ref, min_ref, arg_ref):
    k = pl.program_id(1)

    @pl.when(k == 0)
    def _():
        z = lax.dot_general(w_ref[...], we_ref[...],
                            (((1,), (0,)), ((), ())),
                            preferred_element_type=jnp.float32)
        r_ref[...] = z
        _round_init(z, rm2_ref, rn_ref, min_ref, arg_ref)

    _tile_update(k, cb_ref[0], rm2_ref, rn_ref, min_ref, arg_ref)
    _emit_idx(k, q, idxf_ref, idxi_ref, arg_ref)


def _next_body(q, rp_ref, sel_ref, cb_ref, idxf_ref, idxi_ref, r_ref,
               rm2_ref, rn_ref, min_ref, arg_ref):
    k = pl.program_id(1)

    @pl.when(k == 0)
    def _():
        r = rp_ref[...] - sel_ref[...]
        r_ref[...] = r
        _round_init(r, rm2_ref, rn_ref, min_ref, arg_ref)

    _tile_update(k, cb_ref[0], rm2_ref, rn_ref, min_ref, arg_ref)
    _emit_idx(k, q, idxf_ref, idxi_ref, arg_ref)


def _last_body(q, rp_ref, sel_ref, cb_ref, idxf_ref,
               rm2_ref, rn_ref, min_ref, arg_ref):
    k = pl.program_id(1)

    @pl.when(k == 0)
    def _():
        r = rp_ref[...] - sel_ref[...]
        _round_init(r, rm2_ref, rn_ref, min_ref, arg_ref)

    _tile_update(k, cb_ref[0], rm2_ref, rn_ref, min_ref, arg_ref)
    _emit_idx(k, q, idxf_ref, None, arg_ref)


_SCRATCH = [
    pltpu.VMEM((TB, CODE_DIM), jnp.float32),
    pltpu.VMEM((TB, 1), jnp.float32),
    pltpu.VMEM((TB, 1), jnp.float32),
    pltpu.VMEM((TB, 1), jnp.float32),
]


def _cb_spec(q):
    return pl.BlockSpec((1, KT, CODE_DIM), lambda t, k: (q, k, 0))


_PARAMS = pltpu.CompilerParams(
    dimension_semantics=("parallel", "arbitrary"))


def _tok_spec(cols):
    return pl.BlockSpec((TB, cols), lambda t, k: (t, 0))


def _rep_spec(shape):
    return pl.BlockSpec(shape, lambda t, k: tuple(0 for _ in shape))


_IDX_OUT = [
    jax.ShapeDtypeStruct((N_TOK, 1), jnp.float32),
    jax.ShapeDtypeStruct((N_TOK, 1), jnp.int32),
]


def _vq_first(weights, W_enc, codebooks):
    return pl.pallas_call(
        functools.partial(_first_body, 0),
        grid=(NBLK, NSTEPS),
        in_specs=[_tok_spec(D_IN), _rep_spec((D_IN, CODE_DIM)), _cb_spec(0)],
        out_specs=[_tok_spec(1), _tok_spec(1), _tok_spec(CODE_DIM)],
        out_shape=_IDX_OUT + [
            jax.ShapeDtypeStruct((N_TOK, CODE_DIM), jnp.float32)],
        scratch_shapes=_SCRATCH,
        compiler_params=_PARAMS,
    )(weights, W_enc, codebooks)


def _vq_next(q, r_prev, sel_prev, codebooks):
    return pl.pallas_call(
        functools.partial(_next_body, q),
        grid=(NBLK, NSTEPS),
        in_specs=[_tok_spec(CODE_DIM), _tok_spec(CODE_DIM), _cb_spec(q)],
        out_specs=[_tok_spec(1), _tok_spec(1), _tok_spec(CODE_DIM)],
        out_shape=_IDX_OUT + [
            jax.ShapeDtypeStruct((N_TOK, CODE_DIM), jnp.float32)],
        scratch_shapes=_SCRATCH,
        compiler_params=_PARAMS,
    )(r_prev, sel_prev, codebooks)


def _vq_last(q, r_prev, sel_prev, codebooks):
    return pl.pallas_call(
        functools.partial(_last_body, q),
        grid=(NBLK, NSTEPS),
        in_specs=[_tok_spec(CODE_DIM), _tok_spec(CODE_DIM), _cb_spec(q)],
        out_specs=[_tok_spec(1)],
        out_shape=[jax.ShapeDtypeStruct((N_TOK, 1), jnp.float32)],
        scratch_shapes=_SCRATCH,
        compiler_params=_PARAMS,
    )(r_prev, sel_prev, codebooks)


def _sc_gather(flat_cb, idx):
    info = plsc.get_sparse_core_info()
    nw = info.num_cores * info.num_subcores
    b_per_w = N_TOK // nw
    mesh = plsc.VectorSubcoreMesh(core_axis_name="c", subcore_axis_name="s")

    @functools.partial(
        pl.kernel,
        mesh=mesh,
        out_type=jax.ShapeDtypeStruct((N_TOK, CODE_DIM), jnp.float32),
        scratch_types=[
            pltpu.VMEM((b_per_w,), jnp.int32),
            pltpu.VMEM((b_per_w, CODE_DIM), jnp.float32),
            pltpu.SemaphoreType.DMA,
        ],
    )
    def gather_kernel(table_hbm, idx_hbm, out_hbm, idx_v, rows_v, sem):
        wid = lax.axis_index("s") * info.num_cores + lax.axis_index("c")
        base = wid * b_per_w
        pltpu.sync_copy(idx_hbm.at[pl.ds(base, b_per_w)], idx_v)
        pltpu.async_copy(table_hbm.at[idx_v], rows_v, sem).wait()
        pltpu.sync_copy(rows_v, out_hbm.at[pl.ds(base, b_per_w)])

    return gather_kernel(flat_cb, idx)


def kernel(weights, y, W_enc, codebooks):
    flat_cb = codebooks.reshape(NUM_Q * K, CODE_DIM)
    idxf0, idxi, r = _vq_first(weights, W_enc, codebooks)
    idx_cols = [idxf0]
    for q in range(1, NUM_Q):
        sel = _sc_gather(flat_cb, idxi.reshape(N_TOK))
        if q < NUM_Q - 1:
            idxf, idxi, r = _vq_next(q, r, sel, codebooks)
        else:
            (idxf,) = _vq_last(q, r, sel, codebooks)
        idx_cols.append(idxf)
    indices = jnp.concatenate(idx_cols, axis=1)
    bos = jnp.array([K], dtype=jnp.float32)
    eos = jnp.array([K + 1], dtype=jnp.float32)
    x = jnp.concatenate([bos, indices.reshape(-1), eos])
    return (x, y)
